# baseline (speedup 1.0000x reference)
import functools

import jax
import jax.numpy as jnp
from jax import lax
from jax.experimental import pallas as pl
from jax.experimental.pallas import tpu as pltpu
from jax.experimental.pallas import tpu_sc as plsc

B = 16384
K = 32
W = 128
NUM_CORES = 2
NUM_SUBCORES = 16
NW = NUM_CORES * NUM_SUBCORES
BPW = B // NW
CH = 128
NCH = BPW // CH


def _sc_dev_body(uid_hbm, iid_hbm, jid_hbm, ul_hbm, il_hbm, jl_hbm,
                 user_hbm, item_hbm, dev_hbm,
                 uid_v, iid_v, jid_v, ul_v, il_v, jl_v,
                 u_v, vi_v, vj_v, dev_v, sem):
    cid = lax.axis_index("c")
    sid = lax.axis_index("s")
    wid = sid * NUM_CORES + cid
    base = wid * BPW

    pltpu.sync_copy(uid_hbm.at[pl.ds(base, BPW)], uid_v)
    pltpu.sync_copy(iid_hbm.at[pl.ds(base, BPW)], iid_v)
    pltpu.sync_copy(jid_hbm.at[pl.ds(base, BPW)], jid_v)
    pltpu.sync_copy(ul_hbm.at[pl.ds(base, BPW)], ul_v)
    pltpu.sync_copy(il_hbm.at[pl.ds(base, BPW)], il_v)
    pltpu.sync_copy(jl_hbm.at[pl.ds(base, BPW)], jl_v)

    lane = lax.iota(jnp.int32, 16)

    def chunk_body(c, carry):
        cbase = c * CH
        sl = pl.ds(cbase, CH)
        cu = pltpu.async_copy(user_hbm.at[ul_v.at[sl]], u_v, sem)
        ci = pltpu.async_copy(item_hbm.at[il_v.at[sl]], vi_v, sem)
        cj = pltpu.async_copy(item_hbm.at[jl_v.at[sl]], vj_v, sem)
        cu.wait()
        ci.wait()
        cj.wait()

        def group_body(g, gcarry):
            row = lane + g * 16
            uoff = (uid_v[pl.ds(cbase + g * 16, 16)] & 3) << 5
            ioff = (iid_v[pl.ds(cbase + g * 16, 16)] & 3) << 5
            joff = (jid_v[pl.ds(cbase + g * 16, 16)] & 3) << 5
            acc = jnp.zeros((16,), jnp.float32)
            for k in range(K):
                uk = plsc.load_gather(u_v, [row, uoff + k])
                vik = plsc.load_gather(vi_v, [row, ioff + k])
                vjk = plsc.load_gather(vj_v, [row, joff + k])
                acc = acc + uk * (vik - vjk)
            dev_v[pl.ds(cbase + g * 16, 16)] = acc
            return gcarry

        lax.fori_loop(0, CH // 16, group_body, 0)
        return carry

    lax.fori_loop(0, NCH, chunk_body, 0)

    pltpu.sync_copy(dev_v, dev_hbm.at[pl.ds(base, BPW)])


_sc_dev = functools.partial(
    pl.kernel,
    mesh=plsc.VectorSubcoreMesh(core_axis_name="c", subcore_axis_name="s"),
    out_type=jax.ShapeDtypeStruct((B,), jnp.float32),
    scratch_types=[
        pltpu.VMEM((BPW,), jnp.int32),
        pltpu.VMEM((BPW,), jnp.int32),
        pltpu.VMEM((BPW,), jnp.int32),
        pltpu.VMEM((BPW,), jnp.int32),
        pltpu.VMEM((BPW,), jnp.int32),
        pltpu.VMEM((BPW,), jnp.int32),
        pltpu.VMEM((CH, W), jnp.float32),
        pltpu.VMEM((CH, W), jnp.float32),
        pltpu.VMEM((CH, W), jnp.float32),
        pltpu.VMEM((BPW,), jnp.float32),
        pltpu.SemaphoreType.DMA,
    ],
    compiler_params=pltpu.CompilerParams(
        needs_layout_passes=False, use_tc_tiling_on_sc=False
    ),
)(_sc_dev_body)


def _tc_loss_body(dev_ref, o_ref):
    t = -dev_ref[...]
    sp = jnp.maximum(t, 0.0) + jnp.log1p(jnp.exp(-jnp.abs(t)))
    o_ref[0, 0] = jnp.sum(sp)


def _tc_loss(dev2d):
    return pl.pallas_call(
        _tc_loss_body,
        out_shape=jax.ShapeDtypeStruct((1, 1), jnp.float32),
        in_specs=[pl.BlockSpec(memory_space=pltpu.VMEM)],
        out_specs=pl.BlockSpec(memory_space=pltpu.SMEM),
    )(dev2d)


def kernel(uid, iid, jid, user_matrix, item_matrix):
    uid = uid.astype(jnp.int32)
    iid = iid.astype(jnp.int32)
    jid = jid.astype(jnp.int32)
    user_lines = user_matrix.reshape(user_matrix.shape[0] // 4, W)
    item_lines = item_matrix.reshape(item_matrix.shape[0] // 4, W)
    dev = _sc_dev(uid, iid, jid,
                  jnp.right_shift(uid, 2), jnp.right_shift(iid, 2),
                  jnp.right_shift(jid, 2),
                  user_lines, item_lines)
    loss = _tc_loss(dev.reshape(128, 128))
    return loss[0, 0]


# speedup vs baseline: 1.4876x; 1.4876x over previous
import functools

import jax
import jax.numpy as jnp
from jax import lax
from jax.experimental import pallas as pl
from jax.experimental.pallas import tpu as pltpu
from jax.experimental.pallas import tpu_sc as plsc

B = 16384
K = 32
NUM_CORES = 2
NUM_SUBCORES = 16
NW = NUM_CORES * NUM_SUBCORES
BPW = B // NW
CH = 128


def _sc_dev_body(uid_hbm, iid_hbm, jid_hbm, user_hbm, item_hbm, dev_hbm,
                 uid_s, iid_s, jid_s, u_v, vi_v, vj_v, dev_v, sem, sem_i, sem_j):
    cid = lax.axis_index("c")
    sid = lax.axis_index("s")
    wid = sid * NUM_CORES + cid
    base = wid * BPW

    pltpu.sync_copy(uid_hbm.at[pl.ds(base, BPW)], uid_s)
    pltpu.sync_copy(iid_hbm.at[pl.ds(base, BPW)], iid_s)
    pltpu.sync_copy(jid_hbm.at[pl.ds(base, BPW)], jid_s)

    lane = lax.iota(jnp.int32, 16)

    def chunk_body(c, carry):
        cbase = c * CH

        def fetch_body(g, fcarry):
            idxu = uid_s[pl.ds(cbase + g * 16, 16)]
            idxi = iid_s[pl.ds(cbase + g * 16, 16)]
            idxj = jid_s[pl.ds(cbase + g * 16, 16)]
            for l in range(16):
                e = pl.ds(g * 16 + l, 1)
                pltpu.async_copy(user_hbm.at[pl.ds(idxu[l], 1), :], u_v.at[e, :], sem)
                pltpu.async_copy(item_hbm.at[pl.ds(idxi[l], 1), :], vi_v.at[e, :], sem_i)
                pltpu.async_copy(item_hbm.at[pl.ds(idxj[l], 1), :], vj_v.at[e, :], sem_j)
            return fcarry

        lax.fori_loop(0, CH // 16, fetch_body, 0)

        pltpu.make_async_copy(user_hbm.at[pl.ds(0, CH), :], u_v, sem).wait()
        pltpu.make_async_copy(item_hbm.at[pl.ds(0, CH), :], vi_v, sem_i).wait()
        pltpu.make_async_copy(item_hbm.at[pl.ds(0, CH), :], vj_v, sem_j).wait()

        def group_body(g, gcarry):
            row = lane + g * 16
            acc = jnp.zeros((16,), jnp.float32)
            for k in range(K):
                col = jnp.full((16,), k, jnp.int32)
                uk = plsc.load_gather(u_v, [row, col])
                vik = plsc.load_gather(vi_v, [row, col])
                vjk = plsc.load_gather(vj_v, [row, col])
                acc = acc + uk * (vik - vjk)
            dev_v[pl.ds(cbase + g * 16, 16)] = acc
            return gcarry

        lax.fori_loop(0, CH // 16, group_body, 0)
        return carry

    lax.fori_loop(0, BPW // CH, chunk_body, 0)

    pltpu.sync_copy(dev_v, dev_hbm.at[pl.ds(base, BPW)])


_sc_dev = functools.partial(
    pl.kernel,
    mesh=plsc.VectorSubcoreMesh(core_axis_name="c", subcore_axis_name="s"),
    out_type=jax.ShapeDtypeStruct((B,), jnp.float32),
    scratch_types=[
        pltpu.VMEM((BPW,), jnp.int32),
        pltpu.VMEM((BPW,), jnp.int32),
        pltpu.VMEM((BPW,), jnp.int32),
        pltpu.VMEM((CH, K), jnp.float32),
        pltpu.VMEM((CH, K), jnp.float32),
        pltpu.VMEM((CH, K), jnp.float32),
        pltpu.VMEM((BPW,), jnp.float32),
        pltpu.SemaphoreType.DMA,
        pltpu.SemaphoreType.DMA,
        pltpu.SemaphoreType.DMA,
    ],
    compiler_params=pltpu.CompilerParams(needs_layout_passes=False),
)(_sc_dev_body)


def _tc_loss_body(dev_ref, o_ref):
    t = -dev_ref[...]
    sp = jnp.maximum(t, 0.0) + jnp.log1p(jnp.exp(-jnp.abs(t)))
    o_ref[0, 0] = jnp.sum(sp)


def _tc_loss(dev2d):
    return pl.pallas_call(
        _tc_loss_body,
        out_shape=jax.ShapeDtypeStruct((1, 1), jnp.float32),
        in_specs=[pl.BlockSpec(memory_space=pltpu.VMEM)],
        out_specs=pl.BlockSpec(memory_space=pltpu.SMEM),
    )(dev2d)


def kernel(uid, iid, jid, user_matrix, item_matrix):
    uid = uid.astype(jnp.int32)
    iid = iid.astype(jnp.int32)
    jid = jid.astype(jnp.int32)
    dev = _sc_dev(uid, iid, jid, user_matrix, item_matrix)
    loss = _tc_loss(dev.reshape(128, 128))
    return loss[0, 0]
